# Initial kernel scaffold; baseline (speedup 1.0000x reference)
import jax
import jax.numpy as jnp
from jax.experimental import pallas as pl


def kernel(x, grid):
    raise NotImplementedError("write your pallas kernel here")


# speedup vs baseline: 2.4651x; 2.4651x over previous
import dataclasses
import functools

import jax
import jax.numpy as jnp
from jax import lax
from jax.experimental import pallas as pl
from jax.experimental.pallas import tpu as pltpu
from jax.experimental.pallas import tpu_sc as plsc

N, C, H, W = 4, 192, 224, 224
P = H * W
NP = N * P
ROWS, COLS = 8, NP // 8

NWORKERS = 32
PPW = NP // NWORKERS
B = 64
NCHUNK = PPW // B
NLANE = 16
CSL = C // NLANE


def _index_weight_body(gx_ref, gy_ref, i00, i01, i10, i11, w00, w01, w10, w11):
    gx = gx_ref[...]
    gy = gy_ref[...]
    ix = (gx + 1.0) * (W * 0.5) - 0.5
    iy = (gy + 1.0) * (H * 0.5) - 0.5
    ix0 = jnp.floor(ix)
    iy0 = jnp.floor(iy)
    ix1 = ix0 + 1.0
    iy1 = iy0 + 1.0
    fx = ix - ix0
    fy = iy - iy0

    vx0 = ((ix0 >= 0.0) & (ix0 <= W - 1.0)).astype(jnp.float32)
    vx1 = ((ix1 >= 0.0) & (ix1 <= W - 1.0)).astype(jnp.float32)
    vy0 = ((iy0 >= 0.0) & (iy0 <= H - 1.0)).astype(jnp.float32)
    vy1 = ((iy1 >= 0.0) & (iy1 <= H - 1.0)).astype(jnp.float32)

    cx0 = jnp.clip(ix0, 0.0, W - 1.0).astype(jnp.int32)
    cx1 = jnp.clip(ix1, 0.0, W - 1.0).astype(jnp.int32)
    cy0 = jnp.clip(iy0, 0.0, H - 1.0).astype(jnp.int32)
    cy1 = jnp.clip(iy1, 0.0, H - 1.0).astype(jnp.int32)

    r = lax.broadcasted_iota(jnp.int32, gx.shape, 0)
    base = (r // 2) * P
    i00[...] = base + cy0 * W + cx0
    i01[...] = base + cy0 * W + cx1
    i10[...] = base + cy1 * W + cx0
    i11[...] = base + cy1 * W + cx1
    w00[...] = (1.0 - fx) * (1.0 - fy) * (vx0 * vy0)
    w01[...] = fx * (1.0 - fy) * (vx1 * vy0)
    w10[...] = (1.0 - fx) * fy * (vx0 * vy1)
    w11[...] = fx * fy * (vx1 * vy1)


def _index_weights(gx, gy):
    i32 = jax.ShapeDtypeStruct((ROWS, COLS), jnp.int32)
    f32 = jax.ShapeDtypeStruct((ROWS, COLS), jnp.float32)
    return pl.pallas_call(
        _index_weight_body,
        out_shape=[i32, i32, i32, i32, f32, f32, f32, f32],
    )(gx, gy)


_MESH = plsc.VectorSubcoreMesh(core_axis_name="c", subcore_axis_name="s")

_CP = pltpu.CompilerParams()
if "needs_layout_passes" in pltpu.CompilerParams.__dataclass_fields__:
    _CP = dataclasses.replace(_CP, needs_layout_passes=False)
if "use_tc_tiling_on_sc" in pltpu.CompilerParams.__dataclass_fields__:
    _CP = dataclasses.replace(_CP, use_tc_tiling_on_sc=False)


@functools.partial(
    pl.kernel,
    mesh=_MESH,
    compiler_params=_CP,
    out_type=jax.ShapeDtypeStruct((NP, C), jnp.float32),
    scratch_types=(
        [pltpu.VMEM((B,), jnp.int32) for _ in range(4)]
        + [pltpu.VMEM((B,), jnp.float32) for _ in range(4)]
        + [pltpu.VMEM((B, C), jnp.float32) for _ in range(5)]
        + [pltpu.SemaphoreType.DMA for _ in range(4)]
    ),
)
def _sc_gather_interp(xt, i00, i01, i10, i11, w00, w01, w10, w11, out,
                      iv0, iv1, iv2, iv3, wv0, wv1, wv2, wv3,
                      g0, g1, g2, g3, ov, s0, s1, s2, s3):
    wid = lax.axis_index("s") * 2 + lax.axis_index("c")
    base = wid * PPW

    @pl.loop(0, NCHUNK)
    def _chunk(ci):
        off = base + ci * B
        sl = pl.ds(off, B)
        pltpu.sync_copy(i00.at[sl], iv0)
        pltpu.sync_copy(i01.at[sl], iv1)
        pltpu.sync_copy(i10.at[sl], iv2)
        pltpu.sync_copy(i11.at[sl], iv3)
        pltpu.sync_copy(w00.at[sl], wv0)
        pltpu.sync_copy(w01.at[sl], wv1)
        pltpu.sync_copy(w10.at[sl], wv2)
        pltpu.sync_copy(w11.at[sl], wv3)
        c0 = pltpu.async_copy(xt.at[iv0], g0, s0)
        c1 = pltpu.async_copy(xt.at[iv1], g1, s1)
        c2 = pltpu.async_copy(xt.at[iv2], g2, s2)
        c3 = pltpu.async_copy(xt.at[iv3], g3, s3)
        c0.wait()
        c1.wait()
        c2.wait()
        c3.wait()

        @pl.loop(0, B)
        def _pixel(p):
            pidx = jnp.broadcast_to(p, (NLANE,)).astype(jnp.int32)
            ws0 = plsc.load_gather(wv0, [pidx])
            ws1 = plsc.load_gather(wv1, [pidx])
            ws2 = plsc.load_gather(wv2, [pidx])
            ws3 = plsc.load_gather(wv3, [pidx])
            for j in range(CSL):
                cs = pl.ds(j * NLANE, NLANE)
                acc = (g0[p, cs] * ws0 + g1[p, cs] * ws1
                       + g2[p, cs] * ws2 + g3[p, cs] * ws3)
                ov[p, cs] = acc

        pltpu.sync_copy(ov, out.at[sl])


def kernel(x, grid):
    xt = x.transpose(0, 2, 3, 1).reshape(NP, C)
    gx = grid[..., 0].reshape(ROWS, COLS)
    gy = grid[..., 1].reshape(ROWS, COLS)
    iw = _index_weights(gx, gy)
    flat = [a.reshape(NP) for a in iw]
    out_t = _sc_gather_interp(xt, *flat)
    return out_t.reshape(N, H, W, C).transpose(0, 3, 1, 2)


# speedup vs baseline: 2.9198x; 1.1845x over previous
import dataclasses
import functools

import jax
import jax.numpy as jnp
from jax import lax
from jax.experimental import pallas as pl
from jax.experimental.pallas import tpu as pltpu
from jax.experimental.pallas import tpu_sc as plsc

N, C, H, W = 4, 192, 224, 224
P = H * W
NP = N * P
ROWS, COLS = 8, NP // 8

NWORKERS = 32
PPW = NP // NWORKERS
B = 32
NCH = PPW // B
NLANE = 16
CSL = C // NLANE


def _index_weight_body(gx_ref, gy_ref, iw_ref):
    gx = gx_ref[...]
    gy = gy_ref[...]
    ix = (gx + 1.0) * (W * 0.5) - 0.5
    iy = (gy + 1.0) * (H * 0.5) - 0.5
    ix0 = jnp.floor(ix)
    iy0 = jnp.floor(iy)
    ix1 = ix0 + 1.0
    iy1 = iy0 + 1.0
    fx = ix - ix0
    fy = iy - iy0

    vx0 = ((ix0 >= 0.0) & (ix0 <= W - 1.0)).astype(jnp.float32)
    vx1 = ((ix1 >= 0.0) & (ix1 <= W - 1.0)).astype(jnp.float32)
    vy0 = ((iy0 >= 0.0) & (iy0 <= H - 1.0)).astype(jnp.float32)
    vy1 = ((iy1 >= 0.0) & (iy1 <= H - 1.0)).astype(jnp.float32)

    cx0 = jnp.clip(ix0, 0.0, W - 1.0).astype(jnp.int32)
    cx1 = jnp.clip(ix1, 0.0, W - 1.0).astype(jnp.int32)
    cy0 = jnp.clip(iy0, 0.0, H - 1.0).astype(jnp.int32)
    cy1 = jnp.clip(iy1, 0.0, H - 1.0).astype(jnp.int32)

    r = lax.broadcasted_iota(jnp.int32, gx.shape, 0)
    base = (r // 2) * P
    iw_ref[0] = base + cy0 * W + cx0
    iw_ref[1] = base + cy0 * W + cx1
    iw_ref[2] = base + cy1 * W + cx0
    iw_ref[3] = base + cy1 * W + cx1
    bc = lambda v: lax.bitcast_convert_type(v, jnp.int32)
    iw_ref[4] = bc((1.0 - fx) * (1.0 - fy) * (vx0 * vy0))
    iw_ref[5] = bc(fx * (1.0 - fy) * (vx1 * vy0))
    iw_ref[6] = bc((1.0 - fx) * fy * (vx0 * vy1))
    iw_ref[7] = bc(fx * fy * (vx1 * vy1))


def _index_weights(gx, gy):
    return pl.pallas_call(
        _index_weight_body,
        out_shape=jax.ShapeDtypeStruct((8, ROWS, COLS), jnp.int32),
    )(gx, gy)


_MESH = plsc.VectorSubcoreMesh(core_axis_name="c", subcore_axis_name="s")

_CP = pltpu.CompilerParams()
if "needs_layout_passes" in pltpu.CompilerParams.__dataclass_fields__:
    _CP = dataclasses.replace(_CP, needs_layout_passes=False)
if "use_tc_tiling_on_sc" in pltpu.CompilerParams.__dataclass_fields__:
    _CP = dataclasses.replace(_CP, use_tc_tiling_on_sc=False)


@functools.partial(
    pl.kernel,
    mesh=_MESH,
    compiler_params=_CP,
    out_type=jax.ShapeDtypeStruct((NP, C), jnp.float32),
    scratch_types=(
        [pltpu.VMEM((8, B), jnp.int32) for _ in range(2)]
        + [pltpu.VMEM((B, C), jnp.float32) for _ in range(8)]
        + [pltpu.VMEM((B, C), jnp.float32) for _ in range(2)]
        + [pltpu.SemaphoreType.DMA for _ in range(6)]
    ),
)
def _sc_gather_interp(xt, iw_hbm, out,
                      iwv0, iwv1, g00, g01, g02, g03, g10, g11, g12, g13,
                      ov0, ov1, si0, si1, sg0, sg1, so0, so1):
    iwv = (iwv0, iwv1)
    g = ((g00, g01, g02, g03), (g10, g11, g12, g13))
    ov = (ov0, ov1)
    si = (si0, si1)
    sg = (sg0, sg1)
    so = (so0, so1)

    wid = lax.axis_index("s") * 2 + lax.axis_index("c")
    base = wid * PPW

    def iw_src(ci):
        return iw_hbm.at[:, pl.ds(base + ci * B, B)]

    def out_dst(ci):
        return out.at[pl.ds(base + ci * B, B)]

    def issue_gathers(s):
        for k in range(4):
            pltpu.async_copy(xt.at[iwv[s].at[k]], g[s][k], sg[s])

    def wait_gathers(s):
        for k in range(4):
            pltpu.make_async_copy(xt.at[iwv[s].at[k]], g[s][k], sg[s]).wait()

    def compute(s):
        @pl.loop(0, B)
        def _pixel(p):
            pidx = jnp.broadcast_to(p, (NLANE,)).astype(jnp.int32)
            ws = [
                plsc.bitcast(plsc.load_gather(iwv[s].at[4 + k], [pidx]),
                             jnp.float32)
                for k in range(4)
            ]
            for j in range(CSL):
                cs = pl.ds(j * NLANE, NLANE)
                acc = (g[s][0][p, cs] * ws[0] + g[s][1][p, cs] * ws[1]
                       + g[s][2][p, cs] * ws[2] + g[s][3][p, cs] * ws[3])
                ov[s][p, cs] = acc

    pltpu.sync_copy(iw_src(0), iwv[0])
    issue_gathers(0)
    pltpu.async_copy(iw_src(1), iwv[1], si[1])

    @pl.loop(0, NCH // 2)
    def _it(it):
        ci = it * 2
        for s in (0, 1):
            cc = ci + s
            wait_gathers(s)

            @pl.when(cc + 1 < NCH)
            def _():
                pltpu.make_async_copy(iw_src(cc + 1), iwv[1 - s],
                                      si[1 - s]).wait()
                issue_gathers(1 - s)

            @pl.when(cc >= 2)
            def _():
                pltpu.make_async_copy(ov[s], out_dst(cc - 2), so[s]).wait()

            compute(s)
            pltpu.async_copy(ov[s], out_dst(cc), so[s])

            @pl.when(cc + 2 < NCH)
            def _():
                pltpu.async_copy(iw_src(cc + 2), iwv[s], si[s])

    pltpu.make_async_copy(ov[0], out_dst(NCH - 2), so[0]).wait()
    pltpu.make_async_copy(ov[1], out_dst(NCH - 1), so[1]).wait()


def kernel(x, grid):
    xt = x.transpose(0, 2, 3, 1).reshape(NP, C)
    gx = grid[..., 0].reshape(ROWS, COLS)
    gy = grid[..., 1].reshape(ROWS, COLS)
    iw = _index_weights(gx, gy).reshape(8, NP)
    out_t = _sc_gather_interp(xt, iw)
    return out_t.reshape(N, H, W, C).transpose(0, 3, 1, 2)


# speedup vs baseline: 10.3256x; 3.5365x over previous
import dataclasses
import functools

import jax
import jax.numpy as jnp
from jax import lax
from jax.experimental import pallas as pl
from jax.experimental.pallas import tpu as pltpu
from jax.experimental.pallas import tpu_sc as plsc

N, C, H, W = 4, 192, 224, 224
CPAD = 256
P = H * W
NP = N * P
IROWS, ILANE = NP // 128, 128

NWORKERS = 32
PPW = NP // NWORKERS
B = 32
NCH = PPW // B
NLANE = 16
CSL = C // NLANE

HB = 8
NHB = H // HB


def _index_weight_body(gx_ref, gy_ref, i00, i01, i10, i11,
                       w00, w01, w10, w11):
    gx = gx_ref[...]
    gy = gy_ref[...]
    ix = (gx + 1.0) * (W * 0.5) - 0.5
    iy = (gy + 1.0) * (H * 0.5) - 0.5
    ix0 = jnp.floor(ix)
    iy0 = jnp.floor(iy)
    ix1 = ix0 + 1.0
    iy1 = iy0 + 1.0
    fx = ix - ix0
    fy = iy - iy0

    vx0 = ((ix0 >= 0.0) & (ix0 <= W - 1.0)).astype(jnp.float32)
    vx1 = ((ix1 >= 0.0) & (ix1 <= W - 1.0)).astype(jnp.float32)
    vy0 = ((iy0 >= 0.0) & (iy0 <= H - 1.0)).astype(jnp.float32)
    vy1 = ((iy1 >= 0.0) & (iy1 <= H - 1.0)).astype(jnp.float32)

    cx0 = jnp.clip(ix0, 0.0, W - 1.0).astype(jnp.int32)
    cx1 = jnp.clip(ix1, 0.0, W - 1.0).astype(jnp.int32)
    cy0 = jnp.clip(iy0, 0.0, H - 1.0).astype(jnp.int32)
    cy1 = jnp.clip(iy1, 0.0, H - 1.0).astype(jnp.int32)

    r = lax.broadcasted_iota(jnp.int32, gx.shape, 0)
    base = (r // (P // 128)) * P
    i00[...] = base + cy0 * W + cx0
    i01[...] = base + cy0 * W + cx1
    i10[...] = base + cy1 * W + cx0
    i11[...] = base + cy1 * W + cx1
    w00[...] = (1.0 - fx) * (1.0 - fy) * (vx0 * vy0)
    w01[...] = fx * (1.0 - fy) * (vx1 * vy0)
    w10[...] = (1.0 - fx) * fy * (vx0 * vy1)
    w11[...] = fx * fy * (vx1 * vy1)


def _index_weights(gx, gy):
    i32 = jax.ShapeDtypeStruct((IROWS, ILANE), jnp.int32)
    f32 = jax.ShapeDtypeStruct((IROWS, ILANE), jnp.float32)
    return pl.pallas_call(
        _index_weight_body,
        out_shape=[i32, i32, i32, i32, f32, f32, f32, f32],
    )(gx, gy)


def _tin_body(x_ref, o_ref):
    zpad = jnp.zeros((W, CPAD - C), jnp.float32)
    for i in range(HB):
        o_ref[pl.ds(i * W, W), :] = jnp.concatenate(
            [jnp.transpose(x_ref[0, :, i, :], (1, 0)), zpad], axis=1)


def _transpose_in(x):
    return pl.pallas_call(
        _tin_body,
        grid=(N, NHB),
        in_specs=[pl.BlockSpec((1, C, HB, W), lambda n, j: (n, 0, j, 0))],
        out_specs=pl.BlockSpec((HB * W, CPAD), lambda n, j: (n * NHB + j, 0)),
        out_shape=jax.ShapeDtypeStruct((NP, CPAD), jnp.float32),
    )(x)


def _tout_body(x_ref, o_ref):
    for i in range(HB):
        o_ref[0, :, i, :] = jnp.transpose(
            x_ref[pl.ds(i * W, W), pl.ds(0, C)], (1, 0))


def _transpose_out(out_t):
    return pl.pallas_call(
        _tout_body,
        grid=(N, NHB),
        in_specs=[pl.BlockSpec((HB * W, CPAD), lambda n, j: (n * NHB + j, 0))],
        out_specs=pl.BlockSpec((1, C, HB, W), lambda n, j: (n, 0, j, 0)),
        out_shape=jax.ShapeDtypeStruct((N, C, H, W), jnp.float32),
    )(out_t)


_MESH = plsc.VectorSubcoreMesh(core_axis_name="c", subcore_axis_name="s")

_CP = pltpu.CompilerParams()
if "needs_layout_passes" in pltpu.CompilerParams.__dataclass_fields__:
    _CP = dataclasses.replace(_CP, needs_layout_passes=False)
if "use_tc_tiling_on_sc" in pltpu.CompilerParams.__dataclass_fields__:
    _CP = dataclasses.replace(_CP, use_tc_tiling_on_sc=True)


@functools.partial(
    pl.kernel,
    mesh=_MESH,
    compiler_params=_CP,
    out_type=jax.ShapeDtypeStruct((NP, CPAD), jnp.float32),
    scratch_types=(
        [pltpu.VMEM((B,), jnp.int32) for _ in range(8)]
        + [pltpu.VMEM((B,), jnp.float32) for _ in range(8)]
        + [pltpu.VMEM((B, CPAD), jnp.float32) for _ in range(8)]
        + [pltpu.VMEM((B, CPAD), jnp.float32) for _ in range(2)]
        + [pltpu.SemaphoreType.DMA for _ in range(6)]
    ),
)
def _sc_gather_interp(xt, i00, i01, i10, i11, w00, w01, w10, w11, out,
                      iv00, iv01, iv02, iv03, iv10, iv11, iv12, iv13,
                      wv00, wv01, wv02, wv03, wv10, wv11, wv12, wv13,
                      g00, g01, g02, g03, g10, g11, g12, g13,
                      ov0, ov1, si0, si1, sg0, sg1, so0, so1):
    ihbm = (i00, i01, i10, i11)
    whbm = (w00, w01, w10, w11)
    iv = ((iv00, iv01, iv02, iv03), (iv10, iv11, iv12, iv13))
    wv = ((wv00, wv01, wv02, wv03), (wv10, wv11, wv12, wv13))
    g = ((g00, g01, g02, g03), (g10, g11, g12, g13))
    ov = (ov0, ov1)
    si = (si0, si1)
    sg = (sg0, sg1)
    so = (so0, so1)

    wid = lax.axis_index("s") * 2 + lax.axis_index("c")
    base = wid * PPW

    def issue_iw(ci, s):
        sl = pl.ds(base + ci * B, B)
        for k in range(4):
            pltpu.async_copy(ihbm[k].at[sl], iv[s][k], si[s])
            pltpu.async_copy(whbm[k].at[sl], wv[s][k], si[s])

    def wait_iw(ci, s):
        sl = pl.ds(base + ci * B, B)
        for k in range(4):
            pltpu.make_async_copy(ihbm[k].at[sl], iv[s][k], si[s]).wait()
            pltpu.make_async_copy(whbm[k].at[sl], wv[s][k], si[s]).wait()

    def out_dst(ci):
        return out.at[pl.ds(base + ci * B, B)]

    def issue_gathers(s):
        for k in range(4):
            pltpu.async_copy(xt.at[iv[s][k]], g[s][k], sg[s])

    def wait_gathers(s):
        for k in range(4):
            pltpu.make_async_copy(xt.at[iv[s][k]], g[s][k], sg[s]).wait()

    def compute(s):
        @plsc.parallel_loop(0, B, unroll=2)
        def _pixel(p):
            pidx = jnp.broadcast_to(p, (NLANE,)).astype(jnp.int32)
            ws = [plsc.load_gather(wv[s][k], [pidx]) for k in range(4)]
            for j in range(CSL):
                cs = pl.ds(j * NLANE, NLANE)
                acc = (g[s][0][p, cs] * ws[0] + g[s][1][p, cs] * ws[1]
                       + g[s][2][p, cs] * ws[2] + g[s][3][p, cs] * ws[3])
                ov[s][p, cs] = acc

    sl0 = pl.ds(base, B)
    for k in range(4):
        pltpu.sync_copy(ihbm[k].at[sl0], iv[0][k])
        pltpu.sync_copy(whbm[k].at[sl0], wv[0][k])
    issue_gathers(0)
    issue_iw(1, 1)

    @pl.loop(0, NCH // 2)
    def _it(it):
        ci = it * 2
        for s in (0, 1):
            cc = ci + s
            wait_gathers(s)

            @pl.when(cc + 1 < NCH)
            def _():
                wait_iw(cc + 1, 1 - s)
                issue_gathers(1 - s)

            @pl.when(cc >= 2)
            def _():
                pltpu.make_async_copy(ov[s], out_dst(cc - 2), so[s]).wait()

            compute(s)
            pltpu.async_copy(ov[s], out_dst(cc), so[s])

            @pl.when(cc + 2 < NCH)
            def _():
                issue_iw(cc + 2, s)

    pltpu.make_async_copy(ov[0], out_dst(NCH - 2), so[0]).wait()
    pltpu.make_async_copy(ov[1], out_dst(NCH - 1), so[1]).wait()


def kernel(x, grid):
    xt = _transpose_in(x)
    gx = grid[..., 0].reshape(IROWS, ILANE)
    gy = grid[..., 1].reshape(IROWS, ILANE)
    iw = _index_weights(gx, gy)
    flat = [a.reshape(NP) for a in iw]
    out_t = _sc_gather_interp(xt, *flat)
    return _transpose_out(out_t)


# speedup vs baseline: 11.2039x; 1.0851x over previous
import dataclasses
import functools

import jax
import jax.numpy as jnp
from jax import lax
from jax.experimental import pallas as pl
from jax.experimental.pallas import tpu as pltpu
from jax.experimental.pallas import tpu_sc as plsc

N, C, H, W = 4, 192, 224, 224
CPAD = 256
P = H * W
NP = N * P
IROWS, ILANE = NP // 128, 128

NWORKERS = 32
HP = 2 * P
PPW = HP // NWORKERS
B = 32
NCH = PPW // B
NLANE = 16
CSL = C // NLANE

HB = 8
NHB = H // HB


def _index_weight_body(gx_ref, gy_ref, i00, i01, i10, i11,
                       w00, w01, w10, w11):
    gx = gx_ref[...]
    gy = gy_ref[...]
    ix = (gx + 1.0) * (W * 0.5) - 0.5
    iy = (gy + 1.0) * (H * 0.5) - 0.5
    ix0 = jnp.floor(ix)
    iy0 = jnp.floor(iy)
    ix1 = ix0 + 1.0
    iy1 = iy0 + 1.0
    fx = ix - ix0
    fy = iy - iy0

    vx0 = ((ix0 >= 0.0) & (ix0 <= W - 1.0)).astype(jnp.float32)
    vx1 = ((ix1 >= 0.0) & (ix1 <= W - 1.0)).astype(jnp.float32)
    vy0 = ((iy0 >= 0.0) & (iy0 <= H - 1.0)).astype(jnp.float32)
    vy1 = ((iy1 >= 0.0) & (iy1 <= H - 1.0)).astype(jnp.float32)

    cx0 = jnp.clip(ix0, 0.0, W - 1.0).astype(jnp.int32)
    cx1 = jnp.clip(ix1, 0.0, W - 1.0).astype(jnp.int32)
    cy0 = jnp.clip(iy0, 0.0, H - 1.0).astype(jnp.int32)
    cy1 = jnp.clip(iy1, 0.0, H - 1.0).astype(jnp.int32)

    r = lax.broadcasted_iota(jnp.int32, gx.shape, 0)
    base = ((r // (P // 128)) % 2) * P
    i00[...] = base + cy0 * W + cx0
    i01[...] = base + cy0 * W + cx1
    i10[...] = base + cy1 * W + cx0
    i11[...] = base + cy1 * W + cx1
    w00[...] = (1.0 - fx) * (1.0 - fy) * (vx0 * vy0)
    w01[...] = fx * (1.0 - fy) * (vx1 * vy0)
    w10[...] = (1.0 - fx) * fy * (vx0 * vy1)
    w11[...] = fx * fy * (vx1 * vy1)


def _index_weights(gx, gy):
    i32 = jax.ShapeDtypeStruct((IROWS, ILANE), jnp.int32)
    f32 = jax.ShapeDtypeStruct((IROWS, ILANE), jnp.float32)
    return pl.pallas_call(
        _index_weight_body,
        out_shape=[i32, i32, i32, i32, f32, f32, f32, f32],
    )(gx, gy)


def _tin_body(x_ref, o_ref):
    zpad = jnp.zeros((W, CPAD - C), jnp.float32)
    for i in range(HB):
        o_ref[pl.ds(i * W, W), :] = jnp.concatenate(
            [jnp.transpose(x_ref[0, :, i, :], (1, 0)), zpad], axis=1)


def _transpose_in_half(x, h):
    return pl.pallas_call(
        _tin_body,
        grid=(2, NHB),
        in_specs=[pl.BlockSpec((1, C, HB, W),
                               lambda n, j: (n + 2 * h, 0, j, 0))],
        out_specs=pl.BlockSpec((HB * W, CPAD), lambda n, j: (n * NHB + j, 0)),
        out_shape=jax.ShapeDtypeStruct((2 * P, CPAD), jnp.float32),
    )(x)


def _tout_body(x_ref, o_ref):
    for i in range(HB):
        o_ref[0, :, i, :] = jnp.transpose(
            x_ref[pl.ds(i * W, W), pl.ds(0, C)], (1, 0))


def _tout_body2(x_ref, prev_ref, o_ref):
    del prev_ref
    _tout_body(x_ref, o_ref)


def _transpose_out_first(out_t):
    return pl.pallas_call(
        _tout_body,
        grid=(2, NHB),
        in_specs=[pl.BlockSpec((HB * W, CPAD), lambda n, j: (n * NHB + j, 0))],
        out_specs=pl.BlockSpec((1, C, HB, W), lambda n, j: (n, 0, j, 0)),
        out_shape=jax.ShapeDtypeStruct((N, C, H, W), jnp.float32),
    )(out_t)


def _transpose_out_second(out_t, prev):
    return pl.pallas_call(
        _tout_body2,
        grid=(2, NHB),
        in_specs=[
            pl.BlockSpec((HB * W, CPAD), lambda n, j: (n * NHB + j, 0)),
            pl.BlockSpec(memory_space=pl.ANY),
        ],
        out_specs=pl.BlockSpec((1, C, HB, W), lambda n, j: (n + 2, 0, j, 0)),
        out_shape=jax.ShapeDtypeStruct((N, C, H, W), jnp.float32),
        input_output_aliases={1: 0},
    )(out_t, prev)


_MESH = plsc.VectorSubcoreMesh(core_axis_name="c", subcore_axis_name="s")

_CP = pltpu.CompilerParams()
if "needs_layout_passes" in pltpu.CompilerParams.__dataclass_fields__:
    _CP = dataclasses.replace(_CP, needs_layout_passes=False)
if "use_tc_tiling_on_sc" in pltpu.CompilerParams.__dataclass_fields__:
    _CP = dataclasses.replace(_CP, use_tc_tiling_on_sc=True)


def _make_sc_gather_interp(h):
  @functools.partial(
      pl.kernel,
      mesh=_MESH,
      compiler_params=_CP,
      out_type=jax.ShapeDtypeStruct((HP, CPAD), jnp.float32),
      scratch_types=(
          [pltpu.VMEM((B,), jnp.int32) for _ in range(8)]
          + [pltpu.VMEM((B,), jnp.float32) for _ in range(8)]
          + [pltpu.VMEM((B, CPAD), jnp.float32) for _ in range(8)]
          + [pltpu.VMEM((B, CPAD), jnp.float32) for _ in range(2)]
          + [pltpu.SemaphoreType.DMA for _ in range(6)]
      ),
  )
  def _sc_gather_interp(xt, i00, i01, i10, i11, w00, w01, w10, w11, out,
                        iv00, iv01, iv02, iv03, iv10, iv11, iv12, iv13,
                        wv00, wv01, wv02, wv03, wv10, wv11, wv12, wv13,
                        g00, g01, g02, g03, g10, g11, g12, g13,
                        ov0, ov1, si0, si1, sg0, sg1, so0, so1):
    ihbm = (i00, i01, i10, i11)
    whbm = (w00, w01, w10, w11)
    iv = ((iv00, iv01, iv02, iv03), (iv10, iv11, iv12, iv13))
    wv = ((wv00, wv01, wv02, wv03), (wv10, wv11, wv12, wv13))
    g = ((g00, g01, g02, g03), (g10, g11, g12, g13))
    ov = (ov0, ov1)
    si = (si0, si1)
    sg = (sg0, sg1)
    so = (so0, so1)

    wid = lax.axis_index("s") * 2 + lax.axis_index("c")
    base = wid * PPW
    gbase = h * HP + wid * PPW

    def issue_iw(ci, s):
        sl = pl.ds(gbase + ci * B, B)
        for k in range(4):
            pltpu.async_copy(ihbm[k].at[sl], iv[s][k], si[s])
            pltpu.async_copy(whbm[k].at[sl], wv[s][k], si[s])

    def wait_iw(ci, s):
        sl = pl.ds(gbase + ci * B, B)
        for k in range(4):
            pltpu.make_async_copy(ihbm[k].at[sl], iv[s][k], si[s]).wait()
            pltpu.make_async_copy(whbm[k].at[sl], wv[s][k], si[s]).wait()

    def out_dst(ci):
        return out.at[pl.ds(base + ci * B, B)]

    def issue_gathers(s):
        for k in range(4):
            pltpu.async_copy(xt.at[iv[s][k]], g[s][k], sg[s])

    def wait_gathers(s):
        for k in range(4):
            pltpu.make_async_copy(xt.at[iv[s][k]], g[s][k], sg[s]).wait()

    def compute(s):
        @plsc.parallel_loop(0, B, unroll=2)
        def _pixel(p):
            pidx = jnp.broadcast_to(p, (NLANE,)).astype(jnp.int32)
            ws = [plsc.load_gather(wv[s][k], [pidx]) for k in range(4)]
            for j in range(CSL):
                cs = pl.ds(j * NLANE, NLANE)
                acc = (g[s][0][p, cs] * ws[0] + g[s][1][p, cs] * ws[1]
                       + g[s][2][p, cs] * ws[2] + g[s][3][p, cs] * ws[3])
                ov[s][p, cs] = acc

    sl0 = pl.ds(gbase, B)
    for k in range(4):
        pltpu.sync_copy(ihbm[k].at[sl0], iv[0][k])
        pltpu.sync_copy(whbm[k].at[sl0], wv[0][k])
    issue_gathers(0)
    issue_iw(1, 1)

    @pl.loop(0, NCH // 2)
    def _it(it):
        ci = it * 2
        for s in (0, 1):
            cc = ci + s
            wait_gathers(s)

            @pl.when(cc + 1 < NCH)
            def _():
                wait_iw(cc + 1, 1 - s)
                issue_gathers(1 - s)

            @pl.when(cc >= 2)
            def _():
                pltpu.make_async_copy(ov[s], out_dst(cc - 2), so[s]).wait()

            compute(s)
            pltpu.async_copy(ov[s], out_dst(cc), so[s])

            @pl.when(cc + 2 < NCH)
            def _():
                issue_iw(cc + 2, s)

    pltpu.make_async_copy(ov[0], out_dst(NCH - 2), so[0]).wait()
    pltpu.make_async_copy(ov[1], out_dst(NCH - 1), so[1]).wait()

  return _sc_gather_interp


_SC_HALF = (_make_sc_gather_interp(0), _make_sc_gather_interp(1))


def kernel(x, grid):
    gx = grid[..., 0].reshape(IROWS, ILANE)
    gy = grid[..., 1].reshape(IROWS, ILANE)
    iw = _index_weights(gx, gy)
    flat = [a.reshape(NP) for a in iw]
    xt0 = _transpose_in_half(x, 0)
    out_t0 = _SC_HALF[0](xt0, *flat)
    xt1 = _transpose_in_half(x, 1)
    out_t1 = _SC_HALF[1](xt1, *flat)
    out = _transpose_out_first(out_t0)
    return _transpose_out_second(out_t1, out)


# speedup vs baseline: 11.4300x; 1.0202x over previous
import dataclasses
import functools

import jax
import jax.numpy as jnp
from jax import lax
from jax.experimental import pallas as pl
from jax.experimental.pallas import tpu as pltpu
from jax.experimental.pallas import tpu_sc as plsc

N, C, H, W = 4, 192, 224, 224
CPAD = 256
P = H * W
NP = N * P
IROWS, ILANE = NP // 128, 128

NWORKERS = 32
HP = 2 * P
PPW = HP // NWORKERS
B = 32
NCH = PPW // B
NLANE = 16
CSL = C // NLANE

HB = 16
NHB = H // HB


def _index_weight_body(gx_ref, gy_ref, i00, i01, i10, i11,
                       w00, w01, w10, w11):
    gx = gx_ref[...]
    gy = gy_ref[...]
    ix = (gx + 1.0) * (W * 0.5) - 0.5
    iy = (gy + 1.0) * (H * 0.5) - 0.5
    ix0 = jnp.floor(ix)
    iy0 = jnp.floor(iy)
    ix1 = ix0 + 1.0
    iy1 = iy0 + 1.0
    fx = ix - ix0
    fy = iy - iy0

    vx0 = ((ix0 >= 0.0) & (ix0 <= W - 1.0)).astype(jnp.float32)
    vx1 = ((ix1 >= 0.0) & (ix1 <= W - 1.0)).astype(jnp.float32)
    vy0 = ((iy0 >= 0.0) & (iy0 <= H - 1.0)).astype(jnp.float32)
    vy1 = ((iy1 >= 0.0) & (iy1 <= H - 1.0)).astype(jnp.float32)

    cx0 = jnp.clip(ix0, 0.0, W - 1.0).astype(jnp.int32)
    cx1 = jnp.clip(ix1, 0.0, W - 1.0).astype(jnp.int32)
    cy0 = jnp.clip(iy0, 0.0, H - 1.0).astype(jnp.int32)
    cy1 = jnp.clip(iy1, 0.0, H - 1.0).astype(jnp.int32)

    r = lax.broadcasted_iota(jnp.int32, gx.shape, 0)
    base = ((r // (P // 128)) % 2) * P
    i00[...] = base + cy0 * W + cx0
    i01[...] = base + cy0 * W + cx1
    i10[...] = base + cy1 * W + cx0
    i11[...] = base + cy1 * W + cx1
    w00[...] = (1.0 - fx) * (1.0 - fy) * (vx0 * vy0)
    w01[...] = fx * (1.0 - fy) * (vx1 * vy0)
    w10[...] = (1.0 - fx) * fy * (vx0 * vy1)
    w11[...] = fx * fy * (vx1 * vy1)


def _index_weights(gx, gy):
    i32 = jax.ShapeDtypeStruct((IROWS, ILANE), jnp.int32)
    f32 = jax.ShapeDtypeStruct((IROWS, ILANE), jnp.float32)
    return pl.pallas_call(
        _index_weight_body,
        out_shape=[i32, i32, i32, i32, f32, f32, f32, f32],
    )(gx, gy)


def _tin_body(x_ref, o_ref):
    zpad = jnp.zeros((W, CPAD - C), jnp.float32)
    for i in range(HB):
        o_ref[pl.ds(i * W, W), :] = jnp.concatenate(
            [jnp.transpose(x_ref[0, :, i, :], (1, 0)), zpad], axis=1)


def _transpose_in_half(x, h):
    return pl.pallas_call(
        _tin_body,
        grid=(2, NHB),
        in_specs=[pl.BlockSpec((1, C, HB, W),
                               lambda n, j: (n + 2 * h, 0, j, 0))],
        out_specs=pl.BlockSpec((HB * W, CPAD), lambda n, j: (n * NHB + j, 0)),
        out_shape=jax.ShapeDtypeStruct((2 * P, CPAD), jnp.float32),
    )(x)


def _tout_body(x_ref, o_ref):
    for i in range(HB):
        o_ref[0, :, i, :] = jnp.transpose(
            x_ref[pl.ds(i * W, W), pl.ds(0, C)], (1, 0))


def _tout_body2(x_ref, prev_ref, o_ref):
    del prev_ref
    _tout_body(x_ref, o_ref)


def _transpose_out_first(out_t):
    return pl.pallas_call(
        _tout_body,
        grid=(2, NHB),
        in_specs=[pl.BlockSpec((HB * W, CPAD), lambda n, j: (n * NHB + j, 0))],
        out_specs=pl.BlockSpec((1, C, HB, W), lambda n, j: (n, 0, j, 0)),
        out_shape=jax.ShapeDtypeStruct((N, C, H, W), jnp.float32),
    )(out_t)


def _transpose_out_second(out_t, prev):
    return pl.pallas_call(
        _tout_body2,
        grid=(2, NHB),
        in_specs=[
            pl.BlockSpec((HB * W, CPAD), lambda n, j: (n * NHB + j, 0)),
            pl.BlockSpec(memory_space=pl.ANY),
        ],
        out_specs=pl.BlockSpec((1, C, HB, W), lambda n, j: (n + 2, 0, j, 0)),
        out_shape=jax.ShapeDtypeStruct((N, C, H, W), jnp.float32),
        input_output_aliases={1: 0},
    )(out_t, prev)


_MESH = plsc.VectorSubcoreMesh(core_axis_name="c", subcore_axis_name="s")

_CP = pltpu.CompilerParams()
if "needs_layout_passes" in pltpu.CompilerParams.__dataclass_fields__:
    _CP = dataclasses.replace(_CP, needs_layout_passes=False)
if "use_tc_tiling_on_sc" in pltpu.CompilerParams.__dataclass_fields__:
    _CP = dataclasses.replace(_CP, use_tc_tiling_on_sc=True)


def _make_sc_gather_interp(h):
  @functools.partial(
      pl.kernel,
      mesh=_MESH,
      compiler_params=_CP,
      out_type=jax.ShapeDtypeStruct((HP, CPAD), jnp.float32),
      scratch_types=(
          [pltpu.VMEM((B,), jnp.int32) for _ in range(8)]
          + [pltpu.VMEM((B,), jnp.float32) for _ in range(8)]
          + [pltpu.VMEM((B, CPAD), jnp.float32) for _ in range(8)]
          + [pltpu.VMEM((B, CPAD), jnp.float32) for _ in range(2)]
          + [pltpu.SemaphoreType.DMA for _ in range(6)]
      ),
  )
  def _sc_gather_interp(xt, i00, i01, i10, i11, w00, w01, w10, w11, out,
                        iv00, iv01, iv02, iv03, iv10, iv11, iv12, iv13,
                        wv00, wv01, wv02, wv03, wv10, wv11, wv12, wv13,
                        g00, g01, g02, g03, g10, g11, g12, g13,
                        ov0, ov1, si0, si1, sg0, sg1, so0, so1):
    ihbm = (i00, i01, i10, i11)
    whbm = (w00, w01, w10, w11)
    iv = ((iv00, iv01, iv02, iv03), (iv10, iv11, iv12, iv13))
    wv = ((wv00, wv01, wv02, wv03), (wv10, wv11, wv12, wv13))
    g = ((g00, g01, g02, g03), (g10, g11, g12, g13))
    ov = (ov0, ov1)
    si = (si0, si1)
    sg = (sg0, sg1)
    so = (so0, so1)

    wid = lax.axis_index("s") * 2 + lax.axis_index("c")
    base = wid * PPW
    gbase = h * HP + wid * PPW

    def issue_iw(ci, s):
        sl = pl.ds(gbase + ci * B, B)
        for k in range(4):
            pltpu.async_copy(ihbm[k].at[sl], iv[s][k], si[s])
            pltpu.async_copy(whbm[k].at[sl], wv[s][k], si[s])

    def wait_iw(ci, s):
        sl = pl.ds(gbase + ci * B, B)
        for k in range(4):
            pltpu.make_async_copy(ihbm[k].at[sl], iv[s][k], si[s]).wait()
            pltpu.make_async_copy(whbm[k].at[sl], wv[s][k], si[s]).wait()

    def out_dst(ci):
        return out.at[pl.ds(base + ci * B, B)]

    def issue_gathers(s):
        for k in range(4):
            pltpu.async_copy(xt.at[iv[s][k]], g[s][k], sg[s])

    def wait_gathers(s):
        for k in range(4):
            pltpu.make_async_copy(xt.at[iv[s][k]], g[s][k], sg[s]).wait()

    def compute(s):
        @plsc.parallel_loop(0, B, unroll=2)
        def _pixel(p):
            pidx = jnp.broadcast_to(p, (NLANE,)).astype(jnp.int32)
            ws = [plsc.load_gather(wv[s][k], [pidx]) for k in range(4)]
            for j in range(CSL):
                cs = pl.ds(j * NLANE, NLANE)
                acc = (g[s][0][p, cs] * ws[0] + g[s][1][p, cs] * ws[1]
                       + g[s][2][p, cs] * ws[2] + g[s][3][p, cs] * ws[3])
                ov[s][p, cs] = acc

    sl0 = pl.ds(gbase, B)
    for k in range(4):
        pltpu.sync_copy(ihbm[k].at[sl0], iv[0][k])
        pltpu.sync_copy(whbm[k].at[sl0], wv[0][k])
    issue_gathers(0)
    issue_iw(1, 1)

    @pl.loop(0, NCH // 2)
    def _it(it):
        ci = it * 2
        for s in (0, 1):
            cc = ci + s
            wait_gathers(s)

            @pl.when(cc + 1 < NCH)
            def _():
                wait_iw(cc + 1, 1 - s)
                issue_gathers(1 - s)

            @pl.when(cc >= 2)
            def _():
                pltpu.make_async_copy(ov[s], out_dst(cc - 2), so[s]).wait()

            compute(s)
            pltpu.async_copy(ov[s], out_dst(cc), so[s])

            @pl.when(cc + 2 < NCH)
            def _():
                issue_iw(cc + 2, s)

    pltpu.make_async_copy(ov[0], out_dst(NCH - 2), so[0]).wait()
    pltpu.make_async_copy(ov[1], out_dst(NCH - 1), so[1]).wait()

  return _sc_gather_interp


_SC_HALF = (_make_sc_gather_interp(0), _make_sc_gather_interp(1))


def kernel(x, grid):
    gx = grid[..., 0].reshape(IROWS, ILANE)
    gy = grid[..., 1].reshape(IROWS, ILANE)
    iw = _index_weights(gx, gy)
    flat = [a.reshape(NP) for a in iw]
    xt0 = _transpose_in_half(x, 0)
    out_t0 = _SC_HALF[0](xt0, *flat)
    xt1 = _transpose_in_half(x, 1)
    out_t1 = _SC_HALF[1](xt1, *flat)
    out = _transpose_out_first(out_t0)
    return _transpose_out_second(out_t1, out)
